# baseline (speedup 1.0000x reference)
import functools

import jax
import jax.numpy as jnp
from jax import lax
from jax.experimental import pallas as pl
from jax.experimental.pallas import tpu as pltpu
from jax.experimental.pallas import tpu_sc as plsc

VOCAB = 1000000
D = 64
SEQ = 200
BATCH = 4096
L = 16
NC, NS = 2, 16
NW = NC * NS

N_VT = VOCAB // 128
A_ITERS = (N_VT + NW - 1) // NW
A_PAIRS = (A_ITERS + 1) // 2

B_UNITS = (SEQ // 8) * (BATCH // 512)
B_ITERS = (B_UNITS + NW - 1) // NW
B_PAIRS = (B_ITERS + 1) // 2


def _positional_encoding():
    i = jnp.arange(0, D, 2) / D
    pos = jnp.arange(0, SEQ)[:, None].astype(jnp.float32)
    angle_freq = jnp.exp(i * -jnp.log(jnp.array(10000.0)))
    out = jnp.zeros((SEQ, D), dtype=jnp.float32)
    out = out.at[:, 0::2].set(jnp.sin(pos * angle_freq))
    out = out.at[:, 1::2].set(jnp.cos(pos * angle_freq))
    return out


def _transpose_table(table_t, tail):
    mesh = plsc.VectorSubcoreMesh(core_axis_name="c", subcore_axis_name="s")

    @functools.partial(
        pl.kernel,
        out_type=jax.ShapeDtypeStruct((VOCAB, 128), jnp.float32),
        mesh=mesh,
        compiler_params=pltpu.CompilerParams(needs_layout_passes=False),
        scratch_types=[
            pltpu.VMEM((2, D, 128), jnp.float32),
            pltpu.VMEM((2, 128, 128), jnp.float32),
            pltpu.SemaphoreType.DMA,
            pltpu.SemaphoreType.DMA,
            pltpu.SemaphoreType.DMA,
            pltpu.SemaphoreType.DMA,
        ],
    )
    def body(tt_hbm, tail_hbm, scr_hbm, in_v, st_v, g0, g1, o0, o1):
        wid = lax.axis_index("s") * NC + lax.axis_index("c")
        gsems = (g0, g1)
        osems = (o0, o1)
        n_units = (N_VT - wid + NW - 1) // NW

        def unit_col(k):
            return pl.multiple_of((wid + k * NW) * 128, 128)

        def fire_read(k, nb):
            pltpu.async_copy(
                tt_hbm.at[:, pl.ds(unit_col(k), 128)], in_v.at[nb],
                gsems[nb])

        def handle(k, nb):
            @pl.when(k < n_units)
            def _():
                pltpu.make_async_copy(
                    tt_hbm.at[:, pl.ds(unit_col(k), 128)], in_v.at[nb],
                    gsems[nb]).wait()

                @pl.when(k >= 2)
                def _():
                    pltpu.make_async_copy(
                        st_v.at[nb],
                        scr_hbm.at[pl.ds(unit_col(k - 2), 128), :],
                        osems[nb]).wait()

                def trans_body(t, carry):
                    tvec = jnp.full((L,), t, dtype=jnp.int32)
                    for dg in range(D // L):
                        rows = lax.iota(jnp.int32, L) + (dg * L)
                        v = plsc.load_gather(in_v.at[nb], [rows, tvec])
                        st_v[nb, t, pl.ds(dg * L, L)] = v
                    return carry

                lax.fori_loop(0, 128, trans_body, 0, unroll=4)

                pltpu.async_copy(
                    st_v.at[nb], scr_hbm.at[pl.ds(unit_col(k), 128), :],
                    osems[nb])

            @pl.when(k + 1 < n_units)
            def _():
                fire_read(k + 1, 1 - nb)

        @pl.when(0 < n_units)
        def _():
            fire_read(0, 0)

        def pair_body(p, carry):
            handle(2 * p, 0)
            handle(2 * p + 1, 1)
            return carry

        lax.fori_loop(0, A_PAIRS, pair_body, 0)

        n_even = (n_units + 1) // 2
        n_odd = n_units // 2

        @pl.when(n_even > 0)
        def _():
            pltpu.make_async_copy(
                st_v.at[0], scr_hbm.at[pl.ds(unit_col(0), 128), :],
                osems[0]).wait()

        @pl.when(n_odd > 0)
        def _():
            pltpu.make_async_copy(
                st_v.at[1], scr_hbm.at[pl.ds(unit_col(0), 128), :],
                osems[1]).wait()

        @pl.when(wid == 0)
        def _():
            pltpu.sync_copy(tail_hbm, scr_hbm.at[pl.ds(N_VT * 128, 64), :])

    return body(table_t, tail)


def _gather_embed(idx_t, scr, pos):
    mesh = plsc.VectorSubcoreMesh(core_axis_name="c", subcore_axis_name="s")

    @functools.partial(
        pl.kernel,
        out_type=jax.ShapeDtypeStruct((SEQ, D, BATCH), jnp.float32),
        mesh=mesh,
        compiler_params=pltpu.CompilerParams(needs_layout_passes=False),
        scratch_types=[
            pltpu.VMEM((8, 512), jnp.int32),
            pltpu.VMEM((SEQ, 128), jnp.float32),
            pltpu.VMEM((2, 128, 128), jnp.float32),
            pltpu.VMEM((2, D, 128), jnp.float32),
            pltpu.SemaphoreType.DMA,
            pltpu.SemaphoreType.DMA,
            pltpu.SemaphoreType.DMA,
            pltpu.SemaphoreType.DMA,
        ],
    )
    def body(idx_hbm, scr_hbm, pos_hbm, out_hbm,
             idx_v, pos_v, rows_v, tr_v, g0, g1, o0, o1):
        wid = lax.axis_index("s") * NC + lax.axis_index("c")
        gsems = (g0, g1)
        osems = (o0, o1)
        pltpu.sync_copy(pos_hbm, pos_v)
        n_units = (B_UNITS - wid + NW - 1) // NW

        def stage_unit_idx(k):
            u = wid + k * NW
            s8 = pl.multiple_of((u // 8) * 8, 8)
            b0 = pl.multiple_of((u % 8) * 512, 128)
            pltpu.sync_copy(
                idx_hbm.at[pl.ds(s8, 8), pl.ds(b0, 512)], idx_v)

        def fire_gather(j, nb):
            si = j // 4
            bj = (j % 4) * 128
            pltpu.async_copy(
                scr_hbm.at[idx_v.at[si, pl.ds(bj, 128)]],
                rows_v.at[nb], gsems[nb])

        def sub_block(k, j, nb):
            u = wid + k * NW
            s8 = (u // 8) * 8
            b0 = (u % 8) * 512
            si = j // 4
            s = s8 + si
            bcol = pl.multiple_of(b0 + (j % 4) * 128, 128)

            pltpu.make_async_copy(
                scr_hbm.at[idx_v.at[0, pl.ds(0, 128)]], rows_v.at[nb],
                gsems[nb]).wait()

            @pl.when(j > 1)
            def _():
                pltpu.make_async_copy(
                    tr_v.at[nb], out_hbm.at[0, :, pl.ds(0, 128)],
                    osems[nb]).wait()

            def trans_body(dg, carry):
                pv16 = pos_v[s, pl.ds(dg * L, L)]
                for dl in range(L):
                    d = dg * L + dl
                    pv = jnp.full((L,), pv16[dl], dtype=jnp.float32)
                    dvec = jnp.full((L,), d, dtype=jnp.int32)
                    for tg in range(8):
                        rows = lax.iota(jnp.int32, L) + (tg * L)
                        v = plsc.load_gather(rows_v.at[nb], [rows, dvec])
                        tr_v[nb, d, pl.ds(tg * L, L)] = v + pv
                return carry

            lax.fori_loop(0, D // L, trans_body, 0)

            pltpu.async_copy(
                tr_v.at[nb], out_hbm.at[s, :, pl.ds(bcol, 128)],
                osems[nb])

        def handle_unit(k, carry):
            @pl.when(k < n_units)
            def _():
                stage_unit_idx(k)
                fire_gather(0, 0)

                def sb_pair(p, c2):
                    j0 = 2 * p
                    fire_gather_next(k, j0, 1)
                    sub_block(k, j0, 0)
                    fire_gather_next(k, j0 + 1, 0)
                    sub_block(k, j0 + 1, 1)
                    return c2

                lax.fori_loop(0, 16, sb_pair, 0)

                pltpu.make_async_copy(
                    tr_v.at[0], out_hbm.at[0, :, pl.ds(0, 128)],
                    osems[0]).wait()
                pltpu.make_async_copy(
                    tr_v.at[1], out_hbm.at[0, :, pl.ds(0, 128)],
                    osems[1]).wait()
            return carry

        def fire_gather_next(k, j, nb):
            @pl.when(j + 1 < 32)
            def _():
                fire_gather(j + 1, nb)

        lax.fori_loop(0, B_ITERS, handle_unit, 0)

    return body(idx_t, scr, pos)


def kernel(inputs, table):
    idx_t = inputs.astype(jnp.int32).T
    table_t = table.T
    tail = jnp.pad(table[VOCAB - 64:], ((0, 0), (0, D)))
    pos = jnp.pad(_positional_encoding(), ((0, 0), (0, 128 - D)))
    scr = _transpose_table(table_t, tail)
    out_phys = _gather_embed(idx_t, scr, pos)
    return out_phys.transpose(2, 0, 1)
